# baseline (device time: 392392 ns/iter reference)
import os

import numpy as np

import jax
import jax.numpy as jnp
from jax import lax
from jax.experimental import pallas as pl
from jax.experimental.pallas import tpu as pltpu

N_DEV = 32
B, S, D = 2, 512, 2048
H, Dh, Dr = 16, 128, 32
BS = B * S
ROWS = 2 * BS
CH = BS // N_DEV

_VMEM = pl.BlockSpec(memory_space=pltpu.VMEM)


def _mesh_logical(x, y, z):
    in_plane = {(0, 0): 0, (1, 0): 1, (1, 1): 2, (0, 1): 3,
                (0, 2): 4, (1, 2): 5, (1, 3): 6, (0, 3): 7}[(x, y)]
    return z * 8 + in_plane


def _ring_tables():
    path_yz = []
    for y in range(4):
        zs = range(4) if y % 2 == 0 else range(3, -1, -1)
        path_yz.extend((y, z) for z in zs)
    cycle = [(0, y, z) for (y, z) in path_yz]
    cycle += [(1, y, z) for (y, z) in reversed(path_yz)]
    ring = [_mesh_logical(x, y, z) for (x, y, z) in cycle]
    pos = [0] * N_DEV
    nxt = [0] * N_DEV
    prv = [0] * N_DEV
    for p, l in enumerate(ring):
        pos[l] = p
        nxt[l] = ring[(p + 1) % N_DEV]
        prv[l] = ring[(p - 1) % N_DEV]
    return (np.array(pos, np.int32), np.array(nxt, np.int32),
            np.array(prv, np.int32))


_POS, _NXT, _PRV = _ring_tables()


def _dot(a, b, trans_b=False):
    dn = (((1,), (1 if trans_b else 0,)), ((), ()))
    return lax.dot_general(a, b, dn, preferred_element_type=jnp.float32)


def _partials(xf, wdkv, wuk, wuv):
    def body(x_ref, wdkv_ref, wuk_ref, wuv_ref, kv_ref):
        c = _dot(x_ref[...], wdkv_ref[...])
        kv_ref[0:BS, :] = _dot(c, wuk_ref[...])
        kv_ref[BS:ROWS, :] = _dot(c, wuv_ref[...])

    return pl.pallas_call(
        body,
        out_shape=jax.ShapeDtypeStruct((ROWS, D), jnp.float32),
        in_specs=[_VMEM] * 4,
        out_specs=_VMEM,
    )(xf, wdkv, wuk, wuv)


def _qproj(xf, wq, wqr, wkr):
    def body(x_ref, wq_ref, wqr_ref, wkr_ref, q_ref, qr_ref, kr_ref):
        x = x_ref[...]
        q_ref[...] = _dot(x, wq_ref[...])
        qr_ref[...] = _dot(x, wqr_ref[...])
        kr_ref[...] = _dot(x, wkr_ref[...])

    return pl.pallas_call(
        body,
        out_shape=(
            jax.ShapeDtypeStruct((BS, H * Dh), jnp.float32),
            jax.ShapeDtypeStruct((BS, H * Dr), jnp.float32),
            jax.ShapeDtypeStruct((BS, Dr), jnp.float32),
        ),
        in_specs=[_VMEM] * 4,
        out_specs=(_VMEM, _VMEM, _VMEM),
    )(xf, wq, wqr, wkr)


def _allreduce(kvp, meta):

    def krows(c):
        return pl.ds(c * CH, CH)

    def vrows(c):
        return pl.ds(BS + c * CH, CH)

    def body(meta_ref, in_ref, out_ref, stage_ref,
             ksem_s, ksem_r, vsem_s, vsem_r):
        pos = meta_ref[0]
        nxt = meta_ref[1]
        prv = meta_ref[2]

        barrier = pltpu.get_barrier_semaphore()
        for nbr in (nxt, prv):
            pl.semaphore_signal(
                barrier, inc=1, device_id=(nbr,),
                device_id_type=pl.DeviceIdType.MESH,
            )
        pl.semaphore_wait(barrier, 2)

        out_ref[...] = in_ref[...]

        def start(half, s, src_c, dst_c, dst_space):
            rows = krows if half == "k" else vrows
            sems = (ksem_s, ksem_r) if half == "k" else (vsem_s, vsem_r)
            tgt = nxt if half == "k" else prv
            rdma = pltpu.make_async_remote_copy(
                src_ref=out_ref.at[rows(src_c), :],
                dst_ref=dst_space.at[rows(dst_c), :],
                send_sem=sems[0].at[s],
                recv_sem=sems[1].at[s],
                device_id=(tgt,),
                device_id_type=pl.DeviceIdType.MESH,
            )
            rdma.start()
            return rdma

        for s in range(N_DEV - 1):
            kc = jnp.mod(pos - s, N_DEV)
            vc = jnp.mod(pos + s, N_DEV)
            k_rdma = start("k", s, kc, kc, stage_ref)
            v_rdma = start("v", s, vc, vc, stage_ref)
            k_rdma.wait()
            krc = jnp.mod(pos - s - 1, N_DEV)
            out_ref[krows(krc), :] = (out_ref[krows(krc), :]
                                      + stage_ref[krows(krc), :])
            v_rdma.wait()
            vrc = jnp.mod(pos + s + 1, N_DEV)
            out_ref[vrows(vrc), :] = (out_ref[vrows(vrc), :]
                                      + stage_ref[vrows(vrc), :])

        for s in range(N_DEV - 1):
            kc = jnp.mod(pos + 1 - s, N_DEV)
            vc = jnp.mod(pos - 1 + s, N_DEV)
            k_rdma = start("k", N_DEV - 1 + s, kc, kc, out_ref)
            v_rdma = start("v", N_DEV - 1 + s, vc, vc, out_ref)
            k_rdma.wait()
            v_rdma.wait()

    meta = meta.astype(jnp.int32)
    nsem = 2 * (N_DEV - 1)
    return pl.pallas_call(
        body,
        out_shape=jax.ShapeDtypeStruct((ROWS, D), jnp.float32),
        in_specs=[pl.BlockSpec(memory_space=pltpu.SMEM), _VMEM],
        out_specs=_VMEM,
        scratch_shapes=[
            pltpu.VMEM((ROWS, D), jnp.float32),
            pltpu.SemaphoreType.DMA((nsem,)),
            pltpu.SemaphoreType.DMA((nsem,)),
            pltpu.SemaphoreType.DMA((nsem,)),
            pltpu.SemaphoreType.DMA((nsem,)),
        ],
        compiler_params=pltpu.CompilerParams(collective_id=0),
    )(meta, kvp)


def _attention(kv, q, qr, kr):
    scale = (Dh + Dr) ** -0.5

    def body(kv_ref, q_ref, qr_ref, kr_ref, o_ref):
        for b in range(B):
            rows = slice(b * S, (b + 1) * S)
            kr_b = kr_ref[rows, :]
            for h in range(H):
                cols = slice(h * Dh, (h + 1) * Dh)
                q_bh = q_ref[rows, cols]
                k_bh = kv_ref[rows, cols]
                v_bh = kv_ref[b * S + BS:(b + 1) * S + BS, cols]
                qr_bh = qr_ref[rows, h * Dr:(h + 1) * Dr]
                scores = (_dot(q_bh, k_bh, trans_b=True)
                          + _dot(qr_bh, kr_b, trans_b=True)) * scale
                m = jnp.max(scores, axis=1, keepdims=True)
                p = jnp.exp(scores - m)
                p = p / jnp.sum(p, axis=1, keepdims=True)
                o_ref[rows, cols] = _dot(p, v_bh)

    return pl.pallas_call(
        body,
        out_shape=jax.ShapeDtypeStruct((BS, H * Dh), jnp.float32),
        in_specs=[_VMEM] * 4,
        out_specs=_VMEM,
    )(kv, q, qr, kr)


def _oproj(o, wo):
    def body(o_ref, wo_ref, out_ref):
        out_ref[...] = _dot(o_ref[...], wo_ref[...])

    return pl.pallas_call(
        body,
        out_shape=jax.ShapeDtypeStruct((BS, D), jnp.float32),
        in_specs=[_VMEM, _VMEM],
        out_specs=_VMEM,
    )(o, wo)


def kernel(x, Wdkv, Wuk, Wuv, Wq, Wqr, Wkr, Wo):
    xf = x.reshape(BS, D)
    kvp = _partials(xf, Wdkv, Wuk, Wuv)
    q, qr, kr = _qproj(xf, Wq, Wqr, Wkr)
    if os.environ.get("SKIP_AR"):
        kv = kvp * 32.0
    else:
        me = lax.axis_index("i")
        meta = jnp.stack([
            jnp.asarray(_POS)[me],
            jnp.asarray(_NXT)[me],
            jnp.asarray(_PRV)[me],
        ])
        kv = _allreduce(kvp, meta)
    o = _attention(kv, q, qr, kr)
    out = _oproj(o, Wo)
    return out.reshape(B, S, D)


# device time: 286565 ns/iter; 1.3693x vs baseline; 1.3693x over previous
import os

import numpy as np

import jax
import jax.numpy as jnp
from jax import lax
from jax.experimental import pallas as pl
from jax.experimental.pallas import tpu as pltpu

N_DEV = 32
B, S, D = 2, 512, 2048
H, Dh, Dr = 16, 128, 32
BS = B * S
ROWS = 2 * BS
CH = BS // N_DEV

_VMEM = pl.BlockSpec(memory_space=pltpu.VMEM)


def _mesh_logical(x, y, z):
    in_plane = {(0, 0): 0, (1, 0): 1, (1, 1): 2, (0, 1): 3,
                (0, 2): 4, (1, 2): 5, (1, 3): 6, (0, 3): 7}[(x, y)]
    return z * 8 + in_plane


def _ring_tables():
    path_yz = []
    for y in range(4):
        zs = range(4) if y % 2 == 0 else range(3, -1, -1)
        path_yz.extend((y, z) for z in zs)
    cycle = [(0, y, z) for (y, z) in path_yz]
    cycle += [(1, y, z) for (y, z) in reversed(path_yz)]
    ring = [_mesh_logical(x, y, z) for (x, y, z) in cycle]
    pos = [0] * N_DEV
    nxt = [0] * N_DEV
    prv = [0] * N_DEV
    for p, l in enumerate(ring):
        pos[l] = p
        nxt[l] = ring[(p + 1) % N_DEV]
        prv[l] = ring[(p - 1) % N_DEV]
    return (np.array(pos, np.int32), np.array(nxt, np.int32),
            np.array(prv, np.int32))


_POS, _NXT, _PRV = _ring_tables()


def _dot(a, b, trans_b=False):
    dn = (((1,), (1 if trans_b else 0,)), ((), ()))
    return lax.dot_general(a, b, dn, preferred_element_type=jnp.float32)


def _partials(xf, wdkv, wuk, wuv):
    def body(x_ref, wdkv_ref, wuk_ref, wuv_ref, kv_ref):
        c = _dot(x_ref[...], wdkv_ref[...])
        kv_ref[0:BS, :] = _dot(c, wuk_ref[...])
        kv_ref[BS:ROWS, :] = _dot(c, wuv_ref[...])

    return pl.pallas_call(
        body,
        out_shape=jax.ShapeDtypeStruct((ROWS, D), jnp.float32),
        in_specs=[_VMEM] * 4,
        out_specs=_VMEM,
    )(xf, wdkv, wuk, wuv)


def _qproj(xf, wq, wqr, wkr):
    def body(x_ref, wq_ref, wqr_ref, wkr_ref, q_ref, qr_ref, kr_ref):
        x = x_ref[...]
        q_ref[...] = _dot(x, wq_ref[...])
        qr_ref[...] = _dot(x, wqr_ref[...])
        kr_ref[...] = _dot(x, wkr_ref[...])

    return pl.pallas_call(
        body,
        out_shape=(
            jax.ShapeDtypeStruct((BS, H * Dh), jnp.float32),
            jax.ShapeDtypeStruct((BS, H * Dr), jnp.float32),
            jax.ShapeDtypeStruct((BS, Dr), jnp.float32),
        ),
        in_specs=[_VMEM] * 4,
        out_specs=(_VMEM, _VMEM, _VMEM),
    )(xf, wq, wqr, wkr)


SUB = CH // 2


def _allreduce(kvp, meta):

    def rows(half, c, j):
        base = 0 if half == "k" else BS
        return pl.ds(base + c * CH + j * SUB, SUB)

    def body(meta_ref, in_ref, out_ref, stage_ref, bbuf_ref,
             ksem_s, ksem_r, vsem_s, vsem_r):
        pos = meta_ref[0]
        nxt = meta_ref[1]
        prv = meta_ref[2]

        barrier = pltpu.get_barrier_semaphore()
        for nbr in (nxt, prv):
            pl.semaphore_signal(
                barrier, inc=1, device_id=(nbr,),
                device_id_type=pl.DeviceIdType.MESH,
            )
        pl.semaphore_wait(barrier, 2)

        out_ref[...] = in_ref[...]

        def mk(half, idx, src_space, src_c, dst_space, dst_c, j):
            sems = (ksem_s, ksem_r) if half == "k" else (vsem_s, vsem_r)
            tgt = nxt if half == "k" else prv
            return pltpu.make_async_remote_copy(
                src_ref=src_space.at[rows(half, src_c, j), :],
                dst_ref=dst_space.at[rows(half, dst_c, j), :],
                send_sem=sems[0].at[idx],
                recv_sem=sems[1].at[idx],
                device_id=(tgt,),
                device_id_type=pl.DeviceIdType.MESH,
            )

        sent = []

        def start(half, idx, src_space, c, dst_space, j):
            d = mk(half, idx, src_space, c, dst_space, c, j)
            d.start()
            sent.append(d)
            while len(sent) > 8:
                sent.pop(0).wait_send()
            return d

        for j in range(2):
            start("k", j, out_ref, jnp.mod(pos, N_DEV), stage_ref, j)
            start("v", j, out_ref, jnp.mod(pos, N_DEV), stage_ref, j)
        for s in range(N_DEV - 1):
            krc = jnp.mod(pos - s - 1, N_DEV)
            vrc = jnp.mod(pos + s + 1, N_DEV)
            for j in range(2):
                idx = 2 * s + j
                mk("k", idx, out_ref, krc, stage_ref, krc, j).wait_recv()
                r = rows("k", krc, j)
                out_ref[r, :] = out_ref[r, :] + stage_ref[r, :]
                if s < N_DEV - 2:
                    start("k", idx + 2, out_ref, krc, stage_ref, j)
                mk("v", idx, out_ref, vrc, stage_ref, vrc, j).wait_recv()
                r = rows("v", vrc, j)
                out_ref[r, :] = out_ref[r, :] + stage_ref[r, :]
                if s < N_DEV - 2:
                    start("v", idx + 2, out_ref, vrc, stage_ref, j)
        for d in sent:
            d.wait_send()
        sent.clear()

        kown = jnp.mod(pos + 1, N_DEV)
        vown = jnp.mod(pos - 1, N_DEV)
        for c, half in ((kown, "k"), (vown, "v")):
            r = rows(half, c, 0)
            bbuf_ref[r, :] = out_ref[r, :].astype(jnp.bfloat16)
            r = rows(half, c, 1)
            bbuf_ref[r, :] = out_ref[r, :].astype(jnp.bfloat16)
        for j in range(2):
            start("k", j, bbuf_ref, kown, bbuf_ref, j)
            start("v", j, bbuf_ref, vown, bbuf_ref, j)
        for s in range(N_DEV - 1):
            krc = jnp.mod(pos - s, N_DEV)
            vrc = jnp.mod(pos + s, N_DEV)
            for j in range(2):
                idx = 2 * s + j
                mk("k", idx, bbuf_ref, krc, bbuf_ref, krc, j).wait_recv()
                if s < N_DEV - 2:
                    start("k", idx + 2, bbuf_ref, krc, bbuf_ref, j)
                r = rows("k", krc, j)
                out_ref[r, :] = bbuf_ref[r, :].astype(jnp.float32)
                mk("v", idx, bbuf_ref, vrc, bbuf_ref, vrc, j).wait_recv()
                if s < N_DEV - 2:
                    start("v", idx + 2, bbuf_ref, vrc, bbuf_ref, j)
                r = rows("v", vrc, j)
                out_ref[r, :] = bbuf_ref[r, :].astype(jnp.float32)
        for d in sent:
            d.wait_send()

    meta = meta.astype(jnp.int32)
    nsem = 2 * (N_DEV - 1)
    return pl.pallas_call(
        body,
        out_shape=jax.ShapeDtypeStruct((ROWS, D), jnp.float32),
        in_specs=[pl.BlockSpec(memory_space=pltpu.SMEM), _VMEM],
        out_specs=_VMEM,
        scratch_shapes=[
            pltpu.VMEM((ROWS, D), jnp.float32),
            pltpu.VMEM((ROWS, D), jnp.bfloat16),
            pltpu.SemaphoreType.DMA((nsem,)),
            pltpu.SemaphoreType.DMA((nsem,)),
            pltpu.SemaphoreType.DMA((nsem,)),
            pltpu.SemaphoreType.DMA((nsem,)),
        ],
        compiler_params=pltpu.CompilerParams(
            collective_id=0, vmem_limit_bytes=60 * 1024 * 1024,
        ),
    )(meta, kvp)


def _attention(kv, q, qr, kr):
    scale = (Dh + Dr) ** -0.5

    def body(kv_ref, q_ref, qr_ref, kr_ref, o_ref):
        for b in range(B):
            rows = slice(b * S, (b + 1) * S)
            kr_b = kr_ref[rows, :]
            for h in range(H):
                cols = slice(h * Dh, (h + 1) * Dh)
                q_bh = q_ref[rows, cols]
                k_bh = kv_ref[rows, cols]
                v_bh = kv_ref[b * S + BS:(b + 1) * S + BS, cols]
                qr_bh = qr_ref[rows, h * Dr:(h + 1) * Dr]
                scores = (_dot(q_bh, k_bh, trans_b=True)
                          + _dot(qr_bh, kr_b, trans_b=True)) * scale
                m = jnp.max(scores, axis=1, keepdims=True)
                p = jnp.exp(scores - m)
                p = p / jnp.sum(p, axis=1, keepdims=True)
                o_ref[rows, cols] = _dot(p, v_bh)

    return pl.pallas_call(
        body,
        out_shape=jax.ShapeDtypeStruct((BS, H * Dh), jnp.float32),
        in_specs=[_VMEM] * 4,
        out_specs=_VMEM,
    )(kv, q, qr, kr)


def _oproj(o, wo):
    def body(o_ref, wo_ref, out_ref):
        out_ref[...] = _dot(o_ref[...], wo_ref[...])

    return pl.pallas_call(
        body,
        out_shape=jax.ShapeDtypeStruct((BS, D), jnp.float32),
        in_specs=[_VMEM, _VMEM],
        out_specs=_VMEM,
    )(o, wo)


def kernel(x, Wdkv, Wuk, Wuv, Wq, Wqr, Wkr, Wo):
    xf = x.reshape(BS, D)
    kvp = _partials(xf, Wdkv, Wuk, Wuv)
    q, qr, kr = _qproj(xf, Wq, Wqr, Wkr)
    if os.environ.get("SKIP_AR"):
        kv = kvp * 32.0
    else:
        me = lax.axis_index("i")
        meta = jnp.stack([
            jnp.asarray(_POS)[me],
            jnp.asarray(_NXT)[me],
            jnp.asarray(_PRV)[me],
        ])
        kv = _allreduce(kvp, meta)
    o = _attention(kv, q, qr, kr)
    out = _oproj(o, Wo)
    return out.reshape(B, S, D)


# device time: 252343 ns/iter; 1.5550x vs baseline; 1.1356x over previous
import os

import numpy as np

import jax
import jax.numpy as jnp
from jax import lax
from jax.experimental import pallas as pl
from jax.experimental.pallas import tpu as pltpu

N_DEV = 32
B, S, D = 2, 512, 2048
H, Dh, Dr = 16, 128, 32
BS = B * S
ROWS = 2 * BS
CH = BS // N_DEV
SUB = CH // 2

_VMEM = pl.BlockSpec(memory_space=pltpu.VMEM)


def _mesh_logical(x, y, z):
    in_plane = {(0, 0): 0, (1, 0): 1, (1, 1): 2, (0, 1): 3,
                (0, 2): 4, (1, 2): 5, (1, 3): 6, (0, 3): 7}[(x, y)]
    return z * 8 + in_plane


def _ring_tables():
    path_yz = []
    for y in range(4):
        zs = range(4) if y % 2 == 0 else range(3, -1, -1)
        path_yz.extend((y, z) for z in zs)
    cycle = [(0, y, z) for (y, z) in path_yz]
    cycle += [(1, y, z) for (y, z) in reversed(path_yz)]
    ring = [_mesh_logical(x, y, z) for (x, y, z) in cycle]
    pos = [0] * N_DEV
    nxt = [0] * N_DEV
    prv = [0] * N_DEV
    for p, l in enumerate(ring):
        pos[l] = p
        nxt[l] = ring[(p + 1) % N_DEV]
        prv[l] = ring[(p - 1) % N_DEV]
    return (np.array(pos, np.int32), np.array(nxt, np.int32),
            np.array(prv, np.int32))


_POS, _NXT, _PRV = _ring_tables()


def _dot(a, b, trans_b=False):
    dn = (((1,), (1 if trans_b else 0,)), ((), ()))
    return lax.dot_general(
        a.astype(jnp.bfloat16), b.astype(jnp.bfloat16), dn,
        preferred_element_type=jnp.float32,
    )


def _proj(xf, wdkv, wuk, wuv, wq, wqr, wkr):
    def body(x_ref, wdkv_ref, wuk_ref, wuv_ref, wq_ref, wqr_ref,
             wkr_ref, kv_ref, q_ref, qr_ref, kr_ref):
        x = x_ref[...]
        c = _dot(x, wdkv_ref[...])
        kv_ref[0:BS, :] = _dot(c, wuk_ref[...]).astype(jnp.bfloat16)
        kv_ref[BS:ROWS, :] = _dot(c, wuv_ref[...]).astype(jnp.bfloat16)
        q_ref[...] = _dot(x, wq_ref[...]).astype(jnp.bfloat16)
        qr_ref[...] = _dot(x, wqr_ref[...]).astype(jnp.bfloat16)
        kr_ref[...] = _dot(x, wkr_ref[...]).astype(jnp.bfloat16)

    return pl.pallas_call(
        body,
        out_shape=(
            jax.ShapeDtypeStruct((ROWS, D), jnp.bfloat16),
            jax.ShapeDtypeStruct((BS, H * Dh), jnp.bfloat16),
            jax.ShapeDtypeStruct((BS, H * Dr), jnp.bfloat16),
            jax.ShapeDtypeStruct((BS, Dr), jnp.bfloat16),
        ),
        in_specs=[_VMEM] * 7,
        out_specs=(_VMEM,) * 4,
        compiler_params=pltpu.CompilerParams(
            vmem_limit_bytes=60 * 1024 * 1024,
        ),
    )(xf, wdkv, wuk, wuv, wq, wqr, wkr)


def _allreduce(kvp, meta):

    def rows(half, c, j):
        base = 0 if half == "k" else BS
        return pl.ds(base + c * CH + j * SUB, SUB)

    def body(meta_ref, in_ref, out_ref, acc_ref, bbuf_ref,
             ksem_s, ksem_r, vsem_s, vsem_r):
        pos = meta_ref[0]
        nxt = meta_ref[1]
        prv = meta_ref[2]

        barrier = pltpu.get_barrier_semaphore()
        for nbr in (nxt, prv):
            pl.semaphore_signal(
                barrier, inc=1, device_id=(nbr,),
                device_id_type=pl.DeviceIdType.MESH,
            )
        pl.semaphore_wait(barrier, 2)

        acc_ref[...] = in_ref[...].astype(jnp.float32)

        def mk(half, idx, src_space, src_c, dst_space, dst_c, j):
            sems = (ksem_s, ksem_r) if half == "k" else (vsem_s, vsem_r)
            tgt = nxt if half == "k" else prv
            return pltpu.make_async_remote_copy(
                src_ref=src_space.at[rows(half, src_c, j), :],
                dst_ref=dst_space.at[rows(half, dst_c, j), :],
                send_sem=sems[0].at[idx],
                recv_sem=sems[1].at[idx],
                device_id=(tgt,),
                device_id_type=pl.DeviceIdType.MESH,
            )

        sent = []

        def start(half, idx, src_space, c, dst_space, j):
            d = mk(half, idx, src_space, c, dst_space, c, j)
            d.start()
            sent.append(d)
            while len(sent) > 8:
                sent.pop(0).wait_send()
            return d

        p0 = jnp.mod(pos, N_DEV)
        for j in range(2):
            start("k", j, in_ref, p0, bbuf_ref, j)
            start("v", j, in_ref, p0, bbuf_ref, j)
        for s in range(N_DEV - 1):
            krc = jnp.mod(pos - s - 1, N_DEV)
            vrc = jnp.mod(pos + s + 1, N_DEV)
            for j in range(2):
                idx = 2 * s + j
                mk("k", idx, in_ref, krc, bbuf_ref, krc, j).wait_recv()
                r = rows("k", krc, j)
                acc_ref[r, :] = (acc_ref[r, :]
                                 + bbuf_ref[r, :].astype(jnp.float32))
                if s < N_DEV - 2:
                    bbuf_ref[r, :] = acc_ref[r, :].astype(jnp.bfloat16)
                    start("k", idx + 2, bbuf_ref, krc, bbuf_ref, j)
                mk("v", idx, in_ref, vrc, bbuf_ref, vrc, j).wait_recv()
                r = rows("v", vrc, j)
                acc_ref[r, :] = (acc_ref[r, :]
                                 + bbuf_ref[r, :].astype(jnp.float32))
                if s < N_DEV - 2:
                    bbuf_ref[r, :] = acc_ref[r, :].astype(jnp.bfloat16)
                    start("v", idx + 2, bbuf_ref, vrc, bbuf_ref, j)
        for d in sent:
            d.wait_send()
        sent.clear()

        kown = jnp.mod(pos + 1, N_DEV)
        vown = jnp.mod(pos - 1, N_DEV)
        for c, half in ((kown, "k"), (vown, "v")):
            for j in range(2):
                r = rows(half, c, j)
                out_ref[r, :] = acc_ref[r, :].astype(jnp.bfloat16)
        for j in range(2):
            start("k", j, out_ref, kown, out_ref, j)
            start("v", j, out_ref, vown, out_ref, j)
        for s in range(N_DEV - 1):
            krc = jnp.mod(pos - s, N_DEV)
            vrc = jnp.mod(pos + s, N_DEV)
            for j in range(2):
                idx = 2 * s + j
                mk("k", idx, out_ref, krc, out_ref, krc, j).wait_recv()
                if s < N_DEV - 2:
                    start("k", idx + 2, out_ref, krc, out_ref, j)
                mk("v", idx, out_ref, vrc, out_ref, vrc, j).wait_recv()
                if s < N_DEV - 2:
                    start("v", idx + 2, out_ref, vrc, out_ref, j)
        for d in sent:
            d.wait_send()

    meta = meta.astype(jnp.int32)
    nsem = 2 * (N_DEV - 1)
    return pl.pallas_call(
        body,
        out_shape=jax.ShapeDtypeStruct((ROWS, D), jnp.bfloat16),
        in_specs=[pl.BlockSpec(memory_space=pltpu.SMEM), _VMEM],
        out_specs=_VMEM,
        scratch_shapes=[
            pltpu.VMEM((ROWS, D), jnp.float32),
            pltpu.VMEM((ROWS, D), jnp.bfloat16),
            pltpu.SemaphoreType.DMA((nsem,)),
            pltpu.SemaphoreType.DMA((nsem,)),
            pltpu.SemaphoreType.DMA((nsem,)),
            pltpu.SemaphoreType.DMA((nsem,)),
        ],
        compiler_params=pltpu.CompilerParams(
            collective_id=0, vmem_limit_bytes=60 * 1024 * 1024,
        ),
    )(meta, kvp)


def _attention(kv, q, qr, kr, wo):
    scale = (Dh + Dr) ** -0.5

    def body(kv_ref, q_ref, qr_ref, kr_ref, wo_ref, out_ref, o_ref):
        for b in range(B):
            rows = slice(b * S, (b + 1) * S)
            kr_b = kr_ref[rows, :]
            for h in range(H):
                cols = slice(h * Dh, (h + 1) * Dh)
                q_bh = q_ref[rows, cols]
                k_bh = kv_ref[rows, cols]
                v_bh = kv_ref[b * S + BS:(b + 1) * S + BS, cols]
                qr_bh = qr_ref[rows, h * Dr:(h + 1) * Dr]
                scores = (_dot(q_bh, k_bh, trans_b=True)
                          + _dot(qr_bh, kr_b, trans_b=True)) * scale
                m = jnp.max(scores, axis=1, keepdims=True)
                p = jnp.exp(scores - m)
                p = p / jnp.sum(p, axis=1, keepdims=True)
                o_ref[rows, cols] = _dot(p, v_bh).astype(jnp.bfloat16)
        out_ref[...] = _dot(o_ref[...], wo_ref[...])

    return pl.pallas_call(
        body,
        out_shape=jax.ShapeDtypeStruct((BS, D), jnp.float32),
        in_specs=[_VMEM] * 5,
        out_specs=_VMEM,
        scratch_shapes=[pltpu.VMEM((BS, H * Dh), jnp.bfloat16)],
        compiler_params=pltpu.CompilerParams(
            vmem_limit_bytes=60 * 1024 * 1024,
        ),
    )(kv, q, qr, kr, wo)


def kernel(x, Wdkv, Wuk, Wuv, Wq, Wqr, Wkr, Wo):
    xf = x.reshape(BS, D)
    kvp, q, qr, kr = _proj(xf, Wdkv, Wuk, Wuv, Wq, Wqr, Wkr)
    if os.environ.get("SKIP_AR"):
        kv = (kvp.astype(jnp.float32) * 32.0).astype(jnp.bfloat16)
    else:
        me = lax.axis_index("i")
        meta = jnp.stack([
            jnp.asarray(_POS)[me],
            jnp.asarray(_NXT)[me],
            jnp.asarray(_PRV)[me],
        ])
        kv = _allreduce(kvp, meta)
    out = _attention(kv, q, qr, kr, Wo)
    return out.reshape(B, S, D)


# device time: 250223 ns/iter; 1.5682x vs baseline; 1.0085x over previous
import os

import numpy as np

import jax
import jax.numpy as jnp
from jax import lax
from jax.experimental import pallas as pl
from jax.experimental.pallas import tpu as pltpu

N_DEV = 32
B, S, D = 2, 512, 2048
H, Dh, Dr = 16, 128, 32
BS = B * S
ROWS = 2 * BS
CH = BS // N_DEV
SUB = CH // 2

_VMEM = pl.BlockSpec(memory_space=pltpu.VMEM)


def _mesh_logical(x, y, z):
    in_plane = {(0, 0): 0, (1, 0): 1, (1, 1): 2, (0, 1): 3,
                (0, 2): 4, (1, 2): 5, (1, 3): 6, (0, 3): 7}[(x, y)]
    return z * 8 + in_plane


def _ring_tables():
    path_yz = []
    for y in range(4):
        zs = range(4) if y % 2 == 0 else range(3, -1, -1)
        path_yz.extend((y, z) for z in zs)
    cycle = [(0, y, z) for (y, z) in path_yz]
    cycle += [(1, y, z) for (y, z) in reversed(path_yz)]
    ring = [_mesh_logical(x, y, z) for (x, y, z) in cycle]
    pos = [0] * N_DEV
    nxt = [0] * N_DEV
    prv = [0] * N_DEV
    for p, l in enumerate(ring):
        pos[l] = p
        nxt[l] = ring[(p + 1) % N_DEV]
        prv[l] = ring[(p - 1) % N_DEV]
    return (np.array(pos, np.int32), np.array(nxt, np.int32),
            np.array(prv, np.int32))


_POS, _NXT, _PRV = _ring_tables()


def _dot(a, b, trans_b=False):
    dn = (((1,), (1 if trans_b else 0,)), ((), ()))
    return lax.dot_general(
        a.astype(jnp.bfloat16), b.astype(jnp.bfloat16), dn,
        preferred_element_type=jnp.float32,
    )


def _proj(xf, wdkv, wuk, wuv, wq, wqr, wkr):
    def body(x_ref, wdkv_ref, wuk_ref, wuv_ref, wq_ref, wqr_ref,
             wkr_ref, kv_ref, q_ref, qr_ref, kr_ref):
        x = x_ref[...]
        c = _dot(x, wdkv_ref[...])
        kv_ref[0:BS, :] = _dot(c, wuk_ref[...]).astype(jnp.bfloat16)
        kv_ref[BS:ROWS, :] = _dot(c, wuv_ref[...]).astype(jnp.bfloat16)
        scale = (Dh + Dr) ** -0.5
        q_ref[...] = (_dot(x, wq_ref[...]) * scale).astype(jnp.bfloat16)
        qr_ref[...] = (_dot(x, wqr_ref[...]) * scale).astype(jnp.bfloat16)
        kr_ref[...] = _dot(x, wkr_ref[...]).astype(jnp.bfloat16)

    return pl.pallas_call(
        body,
        out_shape=(
            jax.ShapeDtypeStruct((ROWS, D), jnp.bfloat16),
            jax.ShapeDtypeStruct((BS, H * Dh), jnp.bfloat16),
            jax.ShapeDtypeStruct((BS, H * Dr), jnp.bfloat16),
            jax.ShapeDtypeStruct((BS, Dr), jnp.bfloat16),
        ),
        in_specs=[_VMEM] * 7,
        out_specs=(_VMEM,) * 4,
        compiler_params=pltpu.CompilerParams(
            vmem_limit_bytes=60 * 1024 * 1024,
        ),
    )(xf, wdkv, wuk, wuv, wq, wqr, wkr)


def _allreduce(kvp, meta):

    def rows(half, c, j):
        base = 0 if half == "k" else BS
        return pl.ds(base + c * CH + j * SUB, SUB)

    def body(meta_ref, in_ref, out_ref, acc_ref, bbuf_ref,
             ksem_s, ksem_r, vsem_s, vsem_r):
        pos = meta_ref[0]
        nxt = meta_ref[1]
        prv = meta_ref[2]

        barrier = pltpu.get_barrier_semaphore()
        for nbr in (nxt, prv):
            pl.semaphore_signal(
                barrier, inc=1, device_id=(nbr,),
                device_id_type=pl.DeviceIdType.MESH,
            )
        pl.semaphore_wait(barrier, 2)

        acc_ref[...] = in_ref[...].astype(jnp.float32)

        def mk(half, idx, src_space, src_c, dst_space, dst_c, j):
            sems = (ksem_s, ksem_r) if half == "k" else (vsem_s, vsem_r)
            tgt = nxt if half == "k" else prv
            return pltpu.make_async_remote_copy(
                src_ref=src_space.at[rows(half, src_c, j), :],
                dst_ref=dst_space.at[rows(half, dst_c, j), :],
                send_sem=sems[0].at[idx],
                recv_sem=sems[1].at[idx],
                device_id=(tgt,),
                device_id_type=pl.DeviceIdType.MESH,
            )

        sent = []

        def start(half, idx, src_space, c, dst_space, j):
            d = mk(half, idx, src_space, c, dst_space, c, j)
            d.start()
            sent.append(d)
            while len(sent) > 8:
                sent.pop(0).wait_send()
            return d

        p0 = jnp.mod(pos, N_DEV)
        for j in range(2):
            start("k", j, in_ref, p0, bbuf_ref, j)
            start("v", j, in_ref, p0, bbuf_ref, j)
        for s in range(N_DEV - 1):
            krc = jnp.mod(pos - s - 1, N_DEV)
            vrc = jnp.mod(pos + s + 1, N_DEV)
            for j in range(2):
                idx = 2 * s + j
                mk("k", idx, in_ref, krc, bbuf_ref, krc, j).wait_recv()
                r = rows("k", krc, j)
                acc_ref[r, :] = (acc_ref[r, :]
                                 + bbuf_ref[r, :].astype(jnp.float32))
                if s < N_DEV - 2:
                    bbuf_ref[r, :] = acc_ref[r, :].astype(jnp.bfloat16)
                    start("k", idx + 2, bbuf_ref, krc, bbuf_ref, j)
                mk("v", idx, in_ref, vrc, bbuf_ref, vrc, j).wait_recv()
                r = rows("v", vrc, j)
                acc_ref[r, :] = (acc_ref[r, :]
                                 + bbuf_ref[r, :].astype(jnp.float32))
                if s < N_DEV - 2:
                    bbuf_ref[r, :] = acc_ref[r, :].astype(jnp.bfloat16)
                    start("v", idx + 2, bbuf_ref, vrc, bbuf_ref, j)
        for d in sent:
            d.wait_send()
        sent.clear()

        kown = jnp.mod(pos + 1, N_DEV)
        vown = jnp.mod(pos - 1, N_DEV)
        for c, half in ((kown, "k"), (vown, "v")):
            for j in range(2):
                r = rows(half, c, j)
                out_ref[r, :] = acc_ref[r, :].astype(jnp.bfloat16)
        for j in range(2):
            start("k", j, out_ref, kown, out_ref, j)
            start("v", j, out_ref, vown, out_ref, j)
        for s in range(N_DEV - 1):
            krc = jnp.mod(pos - s, N_DEV)
            vrc = jnp.mod(pos + s, N_DEV)
            for j in range(2):
                idx = 2 * s + j
                mk("k", idx, out_ref, krc, out_ref, krc, j).wait_recv()
                if s < N_DEV - 2:
                    start("k", idx + 2, out_ref, krc, out_ref, j)
                mk("v", idx, out_ref, vrc, out_ref, vrc, j).wait_recv()
                if s < N_DEV - 2:
                    start("v", idx + 2, out_ref, vrc, out_ref, j)
        for d in sent:
            d.wait_send()

    meta = meta.astype(jnp.int32)
    nsem = 2 * (N_DEV - 1)
    return pl.pallas_call(
        body,
        out_shape=jax.ShapeDtypeStruct((ROWS, D), jnp.bfloat16),
        in_specs=[pl.BlockSpec(memory_space=pltpu.SMEM), _VMEM],
        out_specs=_VMEM,
        scratch_shapes=[
            pltpu.VMEM((ROWS, D), jnp.float32),
            pltpu.VMEM((ROWS, D), jnp.bfloat16),
            pltpu.SemaphoreType.DMA((nsem,)),
            pltpu.SemaphoreType.DMA((nsem,)),
            pltpu.SemaphoreType.DMA((nsem,)),
            pltpu.SemaphoreType.DMA((nsem,)),
        ],
        compiler_params=pltpu.CompilerParams(
            collective_id=0, vmem_limit_bytes=60 * 1024 * 1024,
        ),
    )(meta, kvp)


def _attention(kv, q, qr, kr, wo):
    def body(kv_ref, q_ref, qr_ref, kr_ref, wo_ref, out_ref, o_ref):
        for b in range(B):
            rows = slice(b * S, (b + 1) * S)
            kr_b = kr_ref[rows, :]
            for h in range(H):
                cols = slice(h * Dh, (h + 1) * Dh)
                q_bh = q_ref[rows, cols]
                k_bh = kv_ref[rows, cols]
                v_bh = kv_ref[b * S + BS:(b + 1) * S + BS, cols]
                qr_bh = qr_ref[rows, h * Dr:(h + 1) * Dr]
                scores = (_dot(q_bh, k_bh, trans_b=True)
                          + _dot(qr_bh, kr_b, trans_b=True))
                p = jnp.exp(scores)
                denom = jnp.sum(p, axis=1, keepdims=True)
                o = _dot(p, v_bh) / denom
                o_ref[rows, cols] = o.astype(jnp.bfloat16)
        out_ref[...] = _dot(o_ref[...], wo_ref[...])

    return pl.pallas_call(
        body,
        out_shape=jax.ShapeDtypeStruct((BS, D), jnp.float32),
        in_specs=[_VMEM] * 5,
        out_specs=_VMEM,
        scratch_shapes=[pltpu.VMEM((BS, H * Dh), jnp.bfloat16)],
        compiler_params=pltpu.CompilerParams(
            vmem_limit_bytes=60 * 1024 * 1024,
        ),
    )(kv, q, qr, kr, wo)


def kernel(x, Wdkv, Wuk, Wuv, Wq, Wqr, Wkr, Wo):
    xf = x.reshape(BS, D)
    kvp, q, qr, kr = _proj(xf, Wdkv, Wuk, Wuv, Wq, Wqr, Wkr)
    if os.environ.get("SKIP_AR"):
        kv = (kvp.astype(jnp.float32) * 32.0).astype(jnp.bfloat16)
    else:
        me = lax.axis_index("i")
        meta = jnp.stack([
            jnp.asarray(_POS)[me],
            jnp.asarray(_NXT)[me],
            jnp.asarray(_PRV)[me],
        ])
        kv = _allreduce(kvp, meta)
    out = _attention(kv, q, qr, kr, Wo)
    return out.reshape(B, S, D)


# device time: 247514 ns/iter; 1.5853x vs baseline; 1.0109x over previous
import os

import numpy as np

import jax
import jax.numpy as jnp
from jax import lax
from jax.experimental import pallas as pl
from jax.experimental.pallas import tpu as pltpu

N_DEV = 32
B, S, D = 2, 512, 2048
H, Dh, Dr = 16, 128, 32
BS = B * S
ROWS = 2 * BS
CH = BS // N_DEV
SUB = CH // 2

_VMEM = pl.BlockSpec(memory_space=pltpu.VMEM)


def _mesh_logical(x, y, z):
    in_plane = {(0, 0): 0, (1, 0): 1, (1, 1): 2, (0, 1): 3,
                (0, 2): 4, (1, 2): 5, (1, 3): 6, (0, 3): 7}[(x, y)]
    return z * 8 + in_plane


def _ring_tables():
    path_yz = []
    for y in range(4):
        zs = range(4) if y % 2 == 0 else range(3, -1, -1)
        path_yz.extend((y, z) for z in zs)
    cycle = [(0, y, z) for (y, z) in path_yz]
    cycle += [(1, y, z) for (y, z) in reversed(path_yz)]
    ring = [_mesh_logical(x, y, z) for (x, y, z) in cycle]
    pos = [0] * N_DEV
    nxt = [0] * N_DEV
    prv = [0] * N_DEV
    for p, l in enumerate(ring):
        pos[l] = p
        nxt[l] = ring[(p + 1) % N_DEV]
        prv[l] = ring[(p - 1) % N_DEV]
    return (np.array(pos, np.int32), np.array(nxt, np.int32),
            np.array(prv, np.int32))


_POS, _NXT, _PRV = _ring_tables()


def _dot(a, b, trans_b=False):
    dn = (((1,), (1 if trans_b else 0,)), ((), ()))
    return lax.dot_general(
        a.astype(jnp.bfloat16), b.astype(jnp.bfloat16), dn,
        preferred_element_type=jnp.float32,
    )


def _proj(xf, wdkv, wuk, wuv, wq, wqr, wkr):
    def body(x_ref, wdkv_ref, wuk_ref, wuv_ref, wq_ref, wqr_ref,
             wkr_ref, kv_ref, q_ref, qr_ref, kr_ref):
        x = x_ref[...]
        c = _dot(x, wdkv_ref[...])
        kv_ref[0:BS, :] = _dot(c, wuk_ref[...]).astype(jnp.bfloat16)
        kv_ref[BS:ROWS, :] = _dot(c, wuv_ref[...]).astype(jnp.bfloat16)
        scale = (Dh + Dr) ** -0.5
        q_ref[...] = (_dot(x, wq_ref[...]) * scale).astype(jnp.bfloat16)
        qr_ref[...] = (_dot(x, wqr_ref[...]) * scale).astype(jnp.bfloat16)
        kr_ref[...] = _dot(x, wkr_ref[...]).astype(jnp.bfloat16)

    return pl.pallas_call(
        body,
        out_shape=(
            jax.ShapeDtypeStruct((ROWS, D), jnp.bfloat16),
            jax.ShapeDtypeStruct((BS, H * Dh), jnp.bfloat16),
            jax.ShapeDtypeStruct((BS, H * Dr), jnp.bfloat16),
            jax.ShapeDtypeStruct((BS, Dr), jnp.bfloat16),
        ),
        in_specs=[_VMEM] * 7,
        out_specs=(_VMEM,) * 4,
        compiler_params=pltpu.CompilerParams(
            vmem_limit_bytes=60 * 1024 * 1024,
        ),
    )(xf, wdkv, wuk, wuv, wq, wqr, wkr)


def _allreduce(kvp, meta):

    def rows(half, c, j):
        base = 0 if half == "k" else BS
        return pl.ds(base + c * CH + j * SUB, SUB)

    def body(meta_ref, in_ref, out_ref, bbuf_ref,
             ksem_s, ksem_r, vsem_s, vsem_r):
        pos = meta_ref[0]
        nxt = meta_ref[1]
        prv = meta_ref[2]

        barrier = pltpu.get_barrier_semaphore()
        for nbr in (nxt, prv):
            pl.semaphore_signal(
                barrier, inc=1, device_id=(nbr,),
                device_id_type=pl.DeviceIdType.MESH,
            )
        pl.semaphore_wait(barrier, 2)

        def mk(half, idx, src_space, src_c, dst_space, dst_c, j):
            sems = (ksem_s, ksem_r) if half == "k" else (vsem_s, vsem_r)
            tgt = nxt if half == "k" else prv
            return pltpu.make_async_remote_copy(
                src_ref=src_space.at[rows(half, src_c, j), :],
                dst_ref=dst_space.at[rows(half, dst_c, j), :],
                send_sem=sems[0].at[idx],
                recv_sem=sems[1].at[idx],
                device_id=(tgt,),
                device_id_type=pl.DeviceIdType.MESH,
            )

        sent = []

        def start(half, idx, src_space, c, dst_space, j):
            d = mk(half, idx, src_space, c, dst_space, c, j)
            d.start()
            sent.append(d)
            while len(sent) > 8:
                sent.pop(0).wait_send()
            return d

        p0 = jnp.mod(pos, N_DEV)
        for j in range(2):
            start("k", j, in_ref, p0, bbuf_ref, j)
            start("v", j, in_ref, p0, bbuf_ref, j)
        for s in range(N_DEV - 1):
            krc = jnp.mod(pos - s - 1, N_DEV)
            vrc = jnp.mod(pos + s + 1, N_DEV)
            for j in range(2):
                idx = 2 * s + j
                for half, c in (("k", krc), ("v", vrc)):
                    mk(half, idx, in_ref, c, bbuf_ref, c, j).wait_recv()
                    r = rows(half, c, j)
                    t = (bbuf_ref[r, :].astype(jnp.float32)
                         + in_ref[r, :].astype(jnp.float32))
                    if s < N_DEV - 2:
                        bbuf_ref[r, :] = t.astype(jnp.bfloat16)
                        start(half, idx + 2, bbuf_ref, c, bbuf_ref, j)
                    else:
                        out_ref[r, :] = t.astype(jnp.bfloat16)
        for d in sent:
            d.wait_send()
        sent.clear()

        kown = jnp.mod(pos + 1, N_DEV)
        vown = jnp.mod(pos - 1, N_DEV)
        for j in range(2):
            start("k", j, out_ref, kown, out_ref, j)
            start("v", j, out_ref, vown, out_ref, j)
        for s in range(N_DEV - 1):
            krc = jnp.mod(pos - s, N_DEV)
            vrc = jnp.mod(pos + s, N_DEV)
            for j in range(2):
                idx = 2 * s + j
                mk("k", idx, out_ref, krc, out_ref, krc, j).wait_recv()
                if s < N_DEV - 2:
                    start("k", idx + 2, out_ref, krc, out_ref, j)
                mk("v", idx, out_ref, vrc, out_ref, vrc, j).wait_recv()
                if s < N_DEV - 2:
                    start("v", idx + 2, out_ref, vrc, out_ref, j)
        for d in sent:
            d.wait_send()

    meta = meta.astype(jnp.int32)
    nsem = 2 * (N_DEV - 1)
    return pl.pallas_call(
        body,
        out_shape=jax.ShapeDtypeStruct((ROWS, D), jnp.bfloat16),
        in_specs=[pl.BlockSpec(memory_space=pltpu.SMEM), _VMEM],
        out_specs=_VMEM,
        scratch_shapes=[
            pltpu.VMEM((ROWS, D), jnp.bfloat16),
            pltpu.SemaphoreType.DMA((nsem,)),
            pltpu.SemaphoreType.DMA((nsem,)),
            pltpu.SemaphoreType.DMA((nsem,)),
            pltpu.SemaphoreType.DMA((nsem,)),
        ],
        compiler_params=pltpu.CompilerParams(
            collective_id=0, vmem_limit_bytes=60 * 1024 * 1024,
        ),
    )(meta, kvp)


def _attention(kv, q, qr, kr, wo):
    def body(kv_ref, q_ref, qr_ref, kr_ref, wo_ref, out_ref, o_ref):
        for b in range(B):
            rows = slice(b * S, (b + 1) * S)
            kr_b = kr_ref[rows, :]
            for h in range(H):
                cols = slice(h * Dh, (h + 1) * Dh)
                q_bh = q_ref[rows, cols]
                k_bh = kv_ref[rows, cols]
                v_bh = kv_ref[b * S + BS:(b + 1) * S + BS, cols]
                qr_bh = qr_ref[rows, h * Dr:(h + 1) * Dr]
                scores = (_dot(q_bh, k_bh, trans_b=True)
                          + _dot(qr_bh, kr_b, trans_b=True))
                p = jnp.exp(scores.astype(jnp.bfloat16))
                denom = jnp.sum(p, axis=1, keepdims=True,
                                dtype=jnp.float32)
                o = _dot(p, v_bh) / denom
                o_ref[rows, cols] = o.astype(jnp.bfloat16)
        out_ref[...] = _dot(o_ref[...], wo_ref[...])

    return pl.pallas_call(
        body,
        out_shape=jax.ShapeDtypeStruct((BS, D), jnp.float32),
        in_specs=[_VMEM] * 5,
        out_specs=_VMEM,
        scratch_shapes=[pltpu.VMEM((BS, H * Dh), jnp.bfloat16)],
        compiler_params=pltpu.CompilerParams(
            vmem_limit_bytes=60 * 1024 * 1024,
        ),
    )(kv, q, qr, kr, wo)


def kernel(x, Wdkv, Wuk, Wuv, Wq, Wqr, Wkr, Wo):
    xf = x.reshape(BS, D)
    kvp, q, qr, kr = _proj(xf, Wdkv, Wuk, Wuv, Wq, Wqr, Wkr)
    if os.environ.get("SKIP_AR"):
        kv = (kvp.astype(jnp.float32) * 32.0).astype(jnp.bfloat16)
    else:
        me = lax.axis_index("i")
        meta = jnp.stack([
            jnp.asarray(_POS)[me],
            jnp.asarray(_NXT)[me],
            jnp.asarray(_PRV)[me],
        ])
        kv = _allreduce(kvp, meta)
    out = _attention(kv, q, qr, kr, Wo)
    return out.reshape(B, S, D)


# device time: 218527 ns/iter; 1.7956x vs baseline; 1.1326x over previous
import os

import numpy as np

import jax
import jax.numpy as jnp
from jax import lax
from jax.experimental import pallas as pl
from jax.experimental.pallas import tpu as pltpu

N_DEV = 32
B, S, D = 2, 512, 2048
H, Dh, Dr = 16, 128, 32
BS = B * S
ROWS = 2 * BS
CH = BS // N_DEV
SUB = CH // 2

_VMEM = pl.BlockSpec(memory_space=pltpu.VMEM)


def _mesh_logical(x, y, z):
    in_plane = {(0, 0): 0, (1, 0): 1, (1, 1): 2, (0, 1): 3,
                (0, 2): 4, (1, 2): 5, (1, 3): 6, (0, 3): 7}[(x, y)]
    return z * 8 + in_plane


def _ring_tables():
    path_yz = []
    for y in range(4):
        zs = range(4) if y % 2 == 0 else range(3, -1, -1)
        path_yz.extend((y, z) for z in zs)
    cycle = [(0, y, z) for (y, z) in path_yz]
    cycle += [(1, y, z) for (y, z) in reversed(path_yz)]
    ring = [_mesh_logical(x, y, z) for (x, y, z) in cycle]
    pos = [0] * N_DEV
    nxt = [0] * N_DEV
    prv = [0] * N_DEV
    for p, l in enumerate(ring):
        pos[l] = p
        nxt[l] = ring[(p + 1) % N_DEV]
        prv[l] = ring[(p - 1) % N_DEV]
    return (np.array(pos, np.int32), np.array(nxt, np.int32),
            np.array(prv, np.int32))


_POS, _NXT, _PRV = _ring_tables()


def _dot(a, b, trans_b=False):
    dn = (((1,), (1 if trans_b else 0,)), ((), ()))
    return lax.dot_general(
        a.astype(jnp.bfloat16), b.astype(jnp.bfloat16), dn,
        preferred_element_type=jnp.float32,
    )


def _proj(xf, wdkv, wuk, wuv, wq, wqr, wkr):
    def body(x_ref, wdkv_ref, wuk_ref, wuv_ref, wq_ref, wqr_ref,
             wkr_ref, kv_ref, q_ref, qr_ref, kr_ref):
        x = x_ref[...]
        c = _dot(x, wdkv_ref[...])
        kv_ref[0:BS, :] = _dot(c, wuk_ref[...]).astype(jnp.bfloat16)
        kv_ref[BS:ROWS, :] = _dot(c, wuv_ref[...]).astype(jnp.bfloat16)
        scale = (Dh + Dr) ** -0.5
        q_ref[...] = (_dot(x, wq_ref[...]) * scale).astype(jnp.bfloat16)
        qr_ref[...] = (_dot(x, wqr_ref[...]) * scale).astype(jnp.bfloat16)
        kr_ref[...] = _dot(x, wkr_ref[...]).astype(jnp.bfloat16)

    return pl.pallas_call(
        body,
        out_shape=(
            jax.ShapeDtypeStruct((ROWS, D), jnp.bfloat16),
            jax.ShapeDtypeStruct((BS, H * Dh), jnp.bfloat16),
            jax.ShapeDtypeStruct((BS, H * Dr), jnp.bfloat16),
            jax.ShapeDtypeStruct((BS, Dr), jnp.bfloat16),
        ),
        in_specs=[_VMEM] * 7,
        out_specs=(_VMEM,) * 4,
        compiler_params=pltpu.CompilerParams(
            vmem_limit_bytes=60 * 1024 * 1024,
        ),
    )(xf, wdkv, wuk, wuv, wq, wqr, wkr)


def _allreduce(kvp, meta):

    def rows(half, c, j):
        base = 0 if half == "k" else BS
        return pl.ds(base + c * CH + j * SUB, SUB)

    def body(meta_ref, in_ref, out_ref, bbuf_ref,
             ksem_s, ksem_r, vsem_s, vsem_r):
        pos = meta_ref[0]
        nxt = meta_ref[1]
        prv = meta_ref[2]

        barrier = pltpu.get_barrier_semaphore()
        for nbr in (nxt, prv):
            pl.semaphore_signal(
                barrier, inc=1, device_id=(nbr,),
                device_id_type=pl.DeviceIdType.MESH,
            )
        pl.semaphore_wait(barrier, 2)

        def mk(sel, idx, src_space, src_c, dst_space, dst_c, j,
               half=None):
            data = half or sel
            sems = (ksem_s, ksem_r) if sel == "k" else (vsem_s, vsem_r)
            tgt = nxt if sel == "k" else prv
            return pltpu.make_async_remote_copy(
                src_ref=src_space.at[rows(data, src_c, j), :],
                dst_ref=dst_space.at[rows(data, dst_c, j), :],
                send_sem=sems[0].at[idx],
                recv_sem=sems[1].at[idx],
                device_id=(tgt,),
                device_id_type=pl.DeviceIdType.MESH,
            )

        sent = []

        def start(sel, idx, src_space, c, dst_space, j, half=None):
            d = mk(sel, idx, src_space, c, dst_space, c, j, half=half)
            d.start()
            sent.append(d)
            while len(sent) > 8:
                sent.pop(0).wait_send()
            return d

        p0 = jnp.mod(pos, N_DEV)
        for j in range(2):
            start("k", j, in_ref, p0, bbuf_ref, j)
            start("v", j, in_ref, p0, bbuf_ref, j)
        for s in range(N_DEV - 1):
            krc = jnp.mod(pos - s - 1, N_DEV)
            vrc = jnp.mod(pos + s + 1, N_DEV)
            for j in range(2):
                idx = 2 * s + j
                for half, c in (("k", krc), ("v", vrc)):
                    mk(half, idx, in_ref, c, bbuf_ref, c, j).wait_recv()
                    r = rows(half, c, j)
                    t = (bbuf_ref[r, :].astype(jnp.float32)
                         + in_ref[r, :].astype(jnp.float32))
                    if s < N_DEV - 2:
                        bbuf_ref[r, :] = t.astype(jnp.bfloat16)
                        start(half, idx + 2, bbuf_ref, c, bbuf_ref, j)
                    else:
                        out_ref[r, :] = t.astype(jnp.bfloat16)
        for d in sent:
            d.wait_send()
        sent.clear()

        kown = jnp.mod(pos + 1, N_DEV)
        vown = jnp.mod(pos - 1, N_DEV)
        half_steps = N_DEV // 2
        rev_steps = N_DEV // 2 - 1
        for j in range(2):
            start("k", j, out_ref, kown, out_ref, j)
            start("v", j, out_ref, vown, out_ref, j)
            start("v", 32 + j, out_ref, kown, out_ref, j, half="k")
            start("k", 32 + j, out_ref, vown, out_ref, j, half="v")
        for s in range(half_steps):
            k_fwd = jnp.mod(pos - s, N_DEV)
            v_fwd = jnp.mod(pos + s, N_DEV)
            k_rev = jnp.mod(pos + s + 2, N_DEV)
            v_rev = jnp.mod(pos - s - 2, N_DEV)
            for j in range(2):
                idx = 2 * s + j
                mk("k", idx, out_ref, k_fwd, out_ref, k_fwd,
                   j, half="k").wait_recv()
                if s < half_steps - 1:
                    start("k", idx + 2, out_ref, k_fwd, out_ref, j,
                          half="k")
                mk("v", idx, out_ref, v_fwd, out_ref, v_fwd,
                   j, half="v").wait_recv()
                if s < half_steps - 1:
                    start("v", idx + 2, out_ref, v_fwd, out_ref, j,
                          half="v")
                if s < rev_steps:
                    mk("v", 32 + idx, out_ref, k_rev, out_ref, k_rev,
                       j, half="k").wait_recv()
                    if s < rev_steps - 1:
                        start("v", 32 + idx + 2, out_ref, k_rev,
                              out_ref, j, half="k")
                    mk("k", 32 + idx, out_ref, v_rev, out_ref, v_rev,
                       j, half="v").wait_recv()
                    if s < rev_steps - 1:
                        start("k", 32 + idx + 2, out_ref, v_rev,
                              out_ref, j, half="v")
        for d in sent:
            d.wait_send()

    meta = meta.astype(jnp.int32)
    nsem = 2 * (N_DEV - 1)
    return pl.pallas_call(
        body,
        out_shape=jax.ShapeDtypeStruct((ROWS, D), jnp.bfloat16),
        in_specs=[pl.BlockSpec(memory_space=pltpu.SMEM), _VMEM],
        out_specs=_VMEM,
        scratch_shapes=[
            pltpu.VMEM((ROWS, D), jnp.bfloat16),
            pltpu.SemaphoreType.DMA((nsem,)),
            pltpu.SemaphoreType.DMA((nsem,)),
            pltpu.SemaphoreType.DMA((nsem,)),
            pltpu.SemaphoreType.DMA((nsem,)),
        ],
        compiler_params=pltpu.CompilerParams(
            collective_id=0, vmem_limit_bytes=60 * 1024 * 1024,
        ),
    )(meta, kvp)


def _attention(kv, q, qr, kr, wo):
    def body(kv_ref, q_ref, qr_ref, kr_ref, wo_ref, out_ref, o_ref):
        for b in range(B):
            rows = slice(b * S, (b + 1) * S)
            kr_b = kr_ref[rows, :]
            for h in range(H):
                cols = slice(h * Dh, (h + 1) * Dh)
                q_bh = q_ref[rows, cols]
                k_bh = kv_ref[rows, cols]
                v_bh = kv_ref[b * S + BS:(b + 1) * S + BS, cols]
                qr_bh = qr_ref[rows, h * Dr:(h + 1) * Dr]
                scores = (_dot(q_bh, k_bh, trans_b=True)
                          + _dot(qr_bh, kr_b, trans_b=True))
                p = jnp.exp(scores.astype(jnp.bfloat16))
                denom = jnp.sum(p, axis=1, keepdims=True,
                                dtype=jnp.float32)
                o = _dot(p, v_bh) / denom
                o_ref[rows, cols] = o.astype(jnp.bfloat16)
        out_ref[...] = _dot(o_ref[...], wo_ref[...])

    return pl.pallas_call(
        body,
        out_shape=jax.ShapeDtypeStruct((BS, D), jnp.float32),
        in_specs=[_VMEM] * 5,
        out_specs=_VMEM,
        scratch_shapes=[pltpu.VMEM((BS, H * Dh), jnp.bfloat16)],
        compiler_params=pltpu.CompilerParams(
            vmem_limit_bytes=60 * 1024 * 1024,
        ),
    )(kv, q, qr, kr, wo)


def kernel(x, Wdkv, Wuk, Wuv, Wq, Wqr, Wkr, Wo):
    xf = x.reshape(BS, D)
    kvp, q, qr, kr = _proj(xf, Wdkv, Wuk, Wuv, Wq, Wqr, Wkr)
    if os.environ.get("SKIP_AR"):
        kv = (kvp.astype(jnp.float32) * 32.0).astype(jnp.bfloat16)
    else:
        me = lax.axis_index("i")
        meta = jnp.stack([
            jnp.asarray(_POS)[me],
            jnp.asarray(_NXT)[me],
            jnp.asarray(_PRV)[me],
        ])
        kv = _allreduce(kvp, meta)
    out = _attention(kv, q, qr, kr, Wo)
    return out.reshape(B, S, D)
